# Initial kernel scaffold; baseline (speedup 1.0000x reference)
import jax
import jax.numpy as jnp
from jax.experimental import pallas as pl


def kernel(node, edge_index, edge_attr, batch_ptr, W, b, ln_w, ln_b):
    raise NotImplementedError("write your pallas kernel here")


# speedup vs baseline: 3.6092x; 3.6092x over previous
import functools

import jax
import jax.numpy as jnp
from jax import lax
from jax.experimental import pallas as pl
from jax.experimental.pallas import tpu as pltpu
from jax.experimental.pallas import tpu_sc as plsc

N = 10000
E = 160000
D = 256
KHOPS = 3
NQ = 4
H = D // NQ
NSUB = 16
C = 128
EPT = 10112
NCH = EPT // C
NPAD = 10240
STRIPE = NPAD // NSUB


def _sc_propagate(nodes4, rowp, colp, ewp):
    mesh = plsc.VectorSubcoreMesh(core_axis_name="c", subcore_axis_name="s")

    @functools.partial(
        pl.kernel,
        mesh=mesh,
        out_type=jax.ShapeDtypeStruct((KHOPS, NQ, NPAD, H), jnp.float32),
        compiler_params=pltpu.CompilerParams(needs_layout_passes=False,
                                             use_tc_tiling_on_sc=False),
        scratch_types=[
            pltpu.VMEM((NCH, C), jnp.int32),
            pltpu.VMEM((NCH, C), jnp.int32),
            pltpu.VMEM((NCH, C), jnp.float32),
            pltpu.VMEM((NCH, C), jnp.float32),
            pltpu.VMEM((NPAD,), jnp.float32),
            pltpu.VMEM((C, H), jnp.float32),
            pltpu.VMEM((C, H), jnp.float32),
            pltpu.VMEM_SHARED((NPAD, H), jnp.float32),
            pltpu.VMEM_SHARED((NPAD,), jnp.float32),
            pltpu.SemaphoreType.DMA,
        ],
    )
    def body(nodes_hbm, row_hbm, col_hbm, ew_hbm, out_hbm,
             ridx, cidx, ewv, normv, disv, gbuf, zbuf, acc, deg, sem):
        c = lax.axis_index("c")
        s = lax.axis_index("s")

        pltpu.sync_copy(row_hbm.at[s], ridx)
        pltpu.sync_copy(col_hbm.at[s], cidx)
        pltpu.sync_copy(ew_hbm.at[s], ewv)

        def _z1(i, _):
            disv[pl.ds(i * 16, 16)] = jnp.zeros((16,), jnp.float32)
            return 0
        lax.fori_loop(0, NPAD // 16, _z1, 0)

        def _z2(r, _):
            for q in range(H // 16):
                zbuf[r, pl.ds(q * 16, 16)] = jnp.zeros((16,), jnp.float32)
            return 0
        lax.fori_loop(0, C, _z2, 0)

        base = s * STRIPE
        pltpu.sync_copy(disv.at[pl.ds(base, STRIPE)],
                        deg.at[pl.ds(base, STRIPE)])
        plsc.subcore_barrier()

        def _dloop(j, _):
            pltpu.sync_copy(ewv.at[j], deg.at[cidx.at[j]], add=True)
            return 0
        lax.fori_loop(0, NCH, _dloop, 0)
        plsc.subcore_barrier()

        pltpu.sync_copy(deg, disv)

        def _rloop(i, _):
            sl = pl.ds(i * 16, 16)
            x = disv[sl]
            ii = lax.bitcast_convert_type(x, jnp.int32)
            ii = jnp.int32(0x5F3759DF) - lax.shift_right_arithmetic(ii, 1)
            y = lax.bitcast_convert_type(ii, jnp.float32)
            hx = 0.5 * x
            y = y * (1.5 - hx * y * y)
            y = y * (1.5 - hx * y * y)
            y = y * (1.5 - hx * y * y)
            disv[sl] = jnp.where(x > 0.0, y, 0.0)
            return 0
        lax.fori_loop(0, NPAD // 16, _rloop, 0)

        def _nloop(j, _):
            for q in range(C // 16):
                sl = pl.ds(q * 16, 16)
                rv = ridx[j, sl]
                cv = cidx[j, sl]
                ev = ewv[j, sl]
                dr = plsc.load_gather(disv, [rv])
                dc = plsc.load_gather(disv, [cv])
                normv[j, sl] = dr * ev * dc
            return 0
        lax.fori_loop(0, NCH, _nloop, 0)

        for qh in range(NQ // 2):
            qq = 2 * c + qh
            for k in range(KHOPS):
                for z in range(STRIPE // C):
                    pltpu.sync_copy(zbuf, acc.at[pl.ds(base + z * C, C)])
                plsc.subcore_barrier()

                src = (nodes_hbm.at[qq] if k == 0
                       else out_hbm.at[k - 1, qq])

                def _hloop(j, _):
                    pltpu.async_copy(src.at[ridx.at[j]], gbuf, sem).wait()

                    def _eloop(e, _2):
                        nb = plsc.load_gather(
                            normv,
                            [jnp.full((16,), j, jnp.int32),
                             jnp.full((16,), e, jnp.int32)])
                        for q in range(H // 16):
                            sl = pl.ds(q * 16, 16)
                            gbuf[e, sl] = gbuf[e, sl] * nb
                        return 0
                    lax.fori_loop(0, C, _eloop, 0)

                    pltpu.sync_copy(gbuf, acc.at[cidx.at[j]], add=True)
                    return 0
                lax.fori_loop(0, NCH, _hloop, 0)
                plsc.subcore_barrier()

                pltpu.sync_copy(acc.at[pl.ds(base, STRIPE)],
                                out_hbm.at[k, qq, pl.ds(base, STRIPE)])
                plsc.subcore_barrier()

    return body(nodes4, rowp, colp, ewp)


def _tc_dense(node, hs, W, b, ln_w, ln_b):
    BN = 2000
    w0 = W[0]
    wh = W[1:].reshape(KHOPS * NQ, H, D)
    b2 = b.reshape(1, D)
    lnw2 = ln_w.reshape(1, D)
    lnb2 = ln_b.reshape(1, D)

    def body(x_ref, *rest):
        hrefs = rest[:KHOPS * NQ]
        w0_ref, wh_ref, b_ref, lnw_ref, lnb_ref, o_ref = rest[KHOPS * NQ:]
        x = x_ref[...]
        acc = jnp.dot(x, w0_ref[...], preferred_element_type=jnp.float32)
        for m in range(KHOPS * NQ):
            acc = acc + jnp.dot(hrefs[m][...], wh_ref[m],
                                preferred_element_type=jnp.float32)
        y = x + acc + b_ref[...]
        mu = jnp.mean(y, axis=-1, keepdims=True)
        var = jnp.mean((y - mu) * (y - mu), axis=-1, keepdims=True)
        out = (y - mu) * lax.rsqrt(var + 1e-5) * lnw_ref[...] + lnb_ref[...]
        o_ref[...] = jnp.maximum(out, 0.0)

    grid = (N // BN,)
    hspec = pl.BlockSpec((BN, H), lambda i: (i, 0))
    return pl.pallas_call(
        body,
        grid=grid,
        in_specs=[
            pl.BlockSpec((BN, D), lambda i: (i, 0)),
            *([hspec] * (KHOPS * NQ)),
            pl.BlockSpec((D, D), lambda i: (0, 0)),
            pl.BlockSpec((KHOPS * NQ, H, D), lambda i: (0, 0, 0)),
            pl.BlockSpec((1, D), lambda i: (0, 0)),
            pl.BlockSpec((1, D), lambda i: (0, 0)),
            pl.BlockSpec((1, D), lambda i: (0, 0)),
        ],
        out_specs=pl.BlockSpec((BN, D), lambda i: (i, 0)),
        out_shape=jax.ShapeDtypeStruct((N, D), jnp.float32),
    )(node, *hs, w0, wh, b2, lnw2, lnb2)


def kernel(node, edge_index, edge_attr, batch_ptr, W, b, ln_w, ln_b):
    row = edge_index[0]
    col = edge_index[1]
    pad = NSUB * EPT - E
    rowp = jnp.concatenate([row, jnp.zeros((pad,), jnp.int32)]
                           ).reshape(NSUB, NCH, C)
    colp = jnp.concatenate([col, jnp.full((pad,), N, jnp.int32)]
                           ).reshape(NSUB, NCH, C)
    ewp = jnp.concatenate([edge_attr, jnp.zeros((pad,), jnp.float32)]
                          ).reshape(NSUB, NCH, C)
    nodes4 = node.reshape(N, NQ, H).transpose(1, 0, 2)

    hout = _sc_propagate(nodes4, rowp, colp, ewp)
    hs = [hout[k, q, :N] for k in range(KHOPS) for q in range(NQ)]
    return _tc_dense(node, hs, W, b, ln_w, ln_b)
